# baseline (speedup 1.0000x reference)
import functools

import jax
import jax.numpy as jnp
from jax import lax
from jax.experimental import pallas as pl
from jax.experimental.pallas import tpu as pltpu
from jax.experimental.pallas import tpu_sc as plsc

BINS = 256
LANES = 16
ROWS = 32
COLS = 512
NBUF = 4


def _make_sc_hist(n_rows):
    mesh = plsc.VectorSubcoreMesh(core_axis_name="c", subcore_axis_name="s")
    n_workers = mesh.num_cores * mesh.num_subcores
    rows_per_tile = n_rows // n_workers
    n_chunks = rows_per_tile // ROWS
    assert rows_per_tile * n_workers == n_rows
    assert n_chunks * ROWS == rows_per_tile and n_chunks >= NBUF
    hist_words = (BINS + 1) * LANES

    @functools.partial(
        pl.kernel,
        out_type=jax.ShapeDtypeStruct((2, n_workers, 8, COLS), jnp.float32),
        mesh=mesh,
        compiler_params=pltpu.CompilerParams(
            needs_layout_passes=False,
            use_tc_tiling_on_sc=True,
        ),
        scratch_types=[
            pltpu.VMEM((NBUF, ROWS, COLS), jnp.float32),
            pltpu.VMEM((hist_words,), jnp.float32),
            pltpu.VMEM((hist_words,), jnp.float32),
            pltpu.VMEM((8, COLS), jnp.float32),
            pltpu.SemaphoreType.DMA((NBUF,)),
        ],
    )
    def hist_kernel(src_hbm, tgt_hbm, out_hbm, buf, hist_a, hist_b, stage, sem):
        wid = lax.axis_index("c") * mesh.num_subcores + lax.axis_index("s")
        base = wid * rows_per_tile
        lane = lax.iota(jnp.int32, LANES)
        ones = jnp.full((LANES,), 1.0, jnp.float32)

        def zero_body(k, carry):
            z = jnp.zeros((LANES,), jnp.float32)
            hist_a[pl.ds(k * LANES, LANES)] = z
            hist_b[pl.ds(k * LANES, LANES)] = z
            return carry

        lax.fori_loop(0, BINS + 1, zero_body, 0)

        def inner(b, hist):
            @plsc.parallel_loop(0, ROWS * 4, unroll=2)
            def body(i):
                r = jnp.right_shift(i, 2)
                q = jnp.bitwise_and(i, 3) * (8 * LANES)
                for u in range(8):
                    x = buf[b, r, pl.ds(q + u * LANES, LANES)]
                    bn = (x * float(BINS)).astype(jnp.int32)
                    addr = lax.shift_left(bn, 4) | lane
                    plsc.addupdate_scatter(hist, [addr], ones)

        def process(src, hist):
            for c0 in range(NBUF):
                pltpu.async_copy(src.at[pl.ds(base + c0 * ROWS, ROWS), :],
                                 buf.at[c0], sem.at[c0])

            def chunk(c, carry):
                b = jnp.bitwise_and(c, NBUF - 1)
                pltpu.make_async_copy(src.at[pl.ds(base, ROWS), :],
                                      buf.at[b], sem.at[b]).wait()
                inner(b, hist)
                cn = c + NBUF

                @pl.when(cn < n_chunks)
                def _():
                    pltpu.async_copy(src.at[pl.ds(base + cn * ROWS, ROWS), :],
                                     buf.at[b], sem.at[b])

                return carry

            lax.fori_loop(0, n_chunks, chunk, 0)

        def flush(hist, t):
            def fl(k, carry):
                stage[jnp.right_shift(k, 5),
                      pl.ds(jnp.bitwise_and(k, 31) * LANES, LANES)] = (
                          hist[pl.ds(k * LANES, LANES)])
                return carry

            lax.fori_loop(0, BINS, fl, 0)
            stage[7, pl.ds(31 * LANES, LANES)] = (
                stage[7, pl.ds(31 * LANES, LANES)]
                + hist[pl.ds(BINS * LANES, LANES)])
            pltpu.sync_copy(stage, out_hbm.at[t, wid])

        process(src_hbm, hist_a)
        flush(hist_a, 0)
        process(tgt_hbm, hist_b)
        flush(hist_b, 1)

    return hist_kernel


def _loss_body(p_ref, o_ref):
    p = p_ref[...]
    a0 = jnp.sum(p[0], axis=0)
    a1 = jnp.sum(p[1], axis=0)
    sel = (lax.broadcasted_iota(jnp.int32, (COLS, 32), 0) // LANES ==
           lax.broadcasted_iota(jnp.int32, (COLS, 32), 1)).astype(jnp.float32)
    b0 = jax.lax.dot(a0, sel, preferred_element_type=jnp.float32)
    b1 = jax.lax.dot(a1, sel, preferred_element_type=jnp.float32)
    eps = 1e-8
    sn = b0 / jnp.sum(b0) + eps
    tn = b1 / jnp.sum(b1) + eps
    d = sn - tn
    o_ref[...] = (jnp.sum(d * d) * (1.0 / BINS)).reshape(1, 1)


def kernel(source, target):
    m, r, c = source.shape
    s2d = source.reshape((m * r, c))
    t2d = target.reshape((m * r, c))
    partials = _make_sc_hist(m * r)(s2d, t2d)
    loss = pl.pallas_call(
        _loss_body,
        out_shape=jax.ShapeDtypeStruct((1, 1), jnp.float32),
    )(partials)
    return loss.reshape(())
